# baseline (speedup 1.0000x reference)
import jax
import jax.numpy as jnp
from jax import lax
from jax.experimental import pallas as pl
from jax.experimental.pallas import tpu as pltpu
from jax.experimental.pallas import tpu_sc as plsc

B, N, K = 8, 1024, 16
CIN, COUT, M = 64, 64, 8
NK = N * K
R = B * NK
EPS = 1e-5

TK = 4096
NT = NK // TK
AN = 512
FN = 512

SC_CORES = 2
SC_SUBCORES = 16
SC_WORKERS = SC_CORES * SC_SUBCORES
SC_CH = 128
PER_W = R // SC_WORKERS
SC_NCH = PER_W // SC_CH


def _h_body(x_ref, w_ref, g_ref, b_ref, out_ref):
    w = w_ref[...]
    ys = []
    s = jnp.zeros((CIN, 1), jnp.float32)
    q = jnp.zeros((CIN, 1), jnp.float32)
    for b in range(B):
        y = jnp.dot(w, x_ref[b], preferred_element_type=jnp.float32)
        ys.append(y)
        s = s + jnp.sum(y, axis=1, keepdims=True)
        q = q + jnp.sum(y * y, axis=1, keepdims=True)
    cnt = float(B * N)
    mean = s / cnt
    var = q / cnt - mean * mean
    scale = g_ref[...] * lax.rsqrt(var + EPS)
    shift = b_ref[...] - mean * scale
    for b in range(B):
        z = jnp.maximum(ys[b] * scale + shift, 0.0)
        out_ref[b] = z.T.astype(jnp.bfloat16)


def _compute_hT(x, conv1_w, bn1_g, bn1_b):
    return pl.pallas_call(
        _h_body,
        out_shape=jax.ShapeDtypeStruct((B, N, CIN), jnp.bfloat16),
    )(x, conv1_w, bn1_g.reshape(CIN, 1), bn1_b.reshape(CIN, 1))


def _idx_body(i_ref, o_ref):
    b = pl.program_id(0)
    o_ref[0] = i_ref[0].T + b * N


def _prep_idx(idx):
    return pl.pallas_call(
        _idx_body,
        grid=(B,),
        in_specs=[pl.BlockSpec((1, N, K), lambda b: (b, 0, 0))],
        out_specs=pl.BlockSpec((1, K, N), lambda b: (b, 0, 0)),
        out_shape=jax.ShapeDtypeStruct((B, K, N), jnp.int32),
    )(idx)


def _sc_gather_body(table_hbm, gidx_hbm, out_hbm, idx_v, rows_v, sem):
    wid = lax.axis_index("s") * SC_CORES + lax.axis_index("c")
    base = wid * PER_W

    def chunk(c, carry):
        off = base + c * SC_CH
        pltpu.sync_copy(gidx_hbm.at[pl.ds(off, SC_CH)], idx_v)
        pltpu.async_copy(table_hbm.at[idx_v], rows_v, sem).wait()
        pltpu.sync_copy(rows_v, out_hbm.at[pl.ds(off, SC_CH)])
        return carry

    lax.fori_loop(0, SC_NCH, chunk, 0)


def _gather_rows(table, gidx):
    mesh = plsc.VectorSubcoreMesh(
        core_axis_name="c", subcore_axis_name="s",
        num_cores=SC_CORES, num_subcores=SC_SUBCORES)
    fn = pl.kernel(
        _sc_gather_body,
        out_type=jax.ShapeDtypeStruct((R, CIN), jnp.bfloat16),
        mesh=mesh,
        compiler_params=pltpu.CompilerParams(use_tc_tiling_on_sc=False),
        scratch_types=[
            pltpu.VMEM((SC_CH,), jnp.int32),
            pltpu.VMEM((SC_CH, CIN), jnp.bfloat16),
            pltpu.SemaphoreType.DMA,
        ],
    )
    return fn(table, gidx)


def _layer0_body(s_ref, w_ref, y_ref, st_ref, acc_ref):
    t = pl.program_id(0) * pl.num_programs(1) + pl.program_id(1)
    y = jnp.dot(w_ref[...], s_ref[0], preferred_element_type=jnp.float32)
    y_ref[0] = y

    @pl.when(t == 0)
    def _():
        acc_ref[...] = jnp.zeros_like(acc_ref)

    acc_ref[...] += jnp.concatenate(
        [jnp.sum(y, axis=1, keepdims=True),
         jnp.sum(y * y, axis=1, keepdims=True)], axis=1)

    @pl.when(t == pl.num_programs(0) * pl.num_programs(1) - 1)
    def _():
        st_ref[...] = acc_ref[...]


def _mid_body(y_ref, st_ref, g_ref, b_ref, w_ref, o_ref, sto_ref, acc_ref):
    t = pl.program_id(0) * pl.num_programs(1) + pl.num_programs(1) * 0 + pl.program_id(1)
    cnt = float(R)
    st = st_ref[...]
    mean = st[:, 0:1] / cnt
    var = st[:, 1:2] / cnt - mean * mean
    scale = g_ref[...] * lax.rsqrt(var + EPS)
    shift = b_ref[...] - mean * scale
    z = jnp.maximum(y_ref[0] * scale + shift, 0.0)
    y = jnp.dot(w_ref[...], z, preferred_element_type=jnp.float32)
    o_ref[0] = y

    @pl.when(t == 0)
    def _():
        acc_ref[...] = jnp.zeros_like(acc_ref)

    acc_ref[...] += jnp.concatenate(
        [jnp.sum(y, axis=1, keepdims=True),
         jnp.sum(y * y, axis=1, keepdims=True)], axis=1)

    @pl.when(t == pl.num_programs(0) * pl.num_programs(1) - 1)
    def _():
        sto_ref[...] = acc_ref[...]


def _s4_body(y_ref, st_ref, g_ref, b_ref, w_ref, bias_ref, score_ref):
    cnt = float(R)
    st = st_ref[...]
    mean = st[:, 0:1] / cnt
    var = st[:, 1:2] / cnt - mean * mean
    scale = g_ref[...] * lax.rsqrt(var + EPS)
    shift = b_ref[...] - mean * scale
    z = jnp.maximum(y_ref[0] * scale + shift, 0.0)
    y3 = jnp.dot(w_ref[...], z, preferred_element_type=jnp.float32) + bias_ref[...]
    mx = jnp.max(y3, axis=0, keepdims=True)
    e = jnp.exp(y3 - mx)
    sm = e / jnp.sum(e, axis=0, keepdims=True)
    score_ref[0] = sm.T


def _layer0(xyz, w):
    ci, co = w.shape[1], w.shape[0]
    return pl.pallas_call(
        _layer0_body,
        grid=(B, NT),
        in_specs=[
            pl.BlockSpec((1, ci, TK), lambda b, t: (b, 0, t)),
            pl.BlockSpec((co, ci), lambda b, t: (0, 0)),
        ],
        out_specs=[
            pl.BlockSpec((1, co, TK), lambda b, t: (b, 0, t)),
            pl.BlockSpec((co, 2), lambda b, t: (0, 0)),
        ],
        out_shape=[
            jax.ShapeDtypeStruct((B, co, NK), jnp.float32),
            jax.ShapeDtypeStruct((co, 2), jnp.float32),
        ],
        scratch_shapes=[pltpu.VMEM((co, 2), jnp.float32)],
    )(xyz, w)


def _mid_layer(y, st, g, b, w):
    ci, co = w.shape[1], w.shape[0]
    return pl.pallas_call(
        _mid_body,
        grid=(B, NT),
        in_specs=[
            pl.BlockSpec((1, ci, TK), lambda bb, t: (bb, 0, t)),
            pl.BlockSpec((ci, 2), lambda bb, t: (0, 0)),
            pl.BlockSpec((ci, 1), lambda bb, t: (0, 0)),
            pl.BlockSpec((ci, 1), lambda bb, t: (0, 0)),
            pl.BlockSpec((co, ci), lambda bb, t: (0, 0)),
        ],
        out_specs=[
            pl.BlockSpec((1, co, TK), lambda bb, t: (bb, 0, t)),
            pl.BlockSpec((co, 2), lambda bb, t: (0, 0)),
        ],
        out_shape=[
            jax.ShapeDtypeStruct((B, co, NK), jnp.float32),
            jax.ShapeDtypeStruct((co, 2), jnp.float32),
        ],
        scratch_shapes=[pltpu.VMEM((co, 2), jnp.float32)],
    )(y, st, g.reshape(ci, 1), b.reshape(ci, 1), w)


def _last_layer(y, st, g, b, w, bias):
    ci, co = w.shape[1], w.shape[0]
    return pl.pallas_call(
        _s4_body,
        grid=(B, NT),
        in_specs=[
            pl.BlockSpec((1, ci, TK), lambda bb, t: (bb, 0, t)),
            pl.BlockSpec((ci, 2), lambda bb, t: (0, 0)),
            pl.BlockSpec((ci, 1), lambda bb, t: (0, 0)),
            pl.BlockSpec((ci, 1), lambda bb, t: (0, 0)),
            pl.BlockSpec((co, ci), lambda bb, t: (0, 0)),
            pl.BlockSpec((co, 1), lambda bb, t: (0, 0)),
        ],
        out_specs=pl.BlockSpec((1, TK, co), lambda bb, t: (bb, t, 0)),
        out_shape=jax.ShapeDtypeStruct((B, NK, co), jnp.float32),
    )(y, st, g.reshape(ci, 1), b.reshape(ci, 1), w, bias.reshape(co, 1))


def _agg_body(hg_ref, sc_ref, w2_ref, o_ref, st_ref, acc_ref):
    t = pl.program_id(0) * pl.num_programs(1) + pl.program_id(1)
    hgv = hg_ref[0].astype(jnp.float32)
    scv = jnp.transpose(sc_ref[0], (1, 0, 2))
    acc = jnp.zeros((AN, COUT), jnp.float32)
    for m in range(M):
        w = scv[:, :, m:m + 1]
        am = jnp.sum(hgv * w, axis=0)
        acc = acc + jnp.dot(am, w2_ref[m], preferred_element_type=jnp.float32)
    o_ref[0] = acc

    @pl.when(t == 0)
    def _():
        acc_ref[...] = jnp.zeros_like(acc_ref)

    acc_ref[...] += jnp.concatenate(
        [jnp.sum(acc, axis=0, keepdims=True),
         jnp.sum(acc * acc, axis=0, keepdims=True)], axis=0)

    @pl.when(t == pl.num_programs(0) * pl.num_programs(1) - 1)
    def _():
        st_ref[...] = acc_ref[...]


def _aggregate(hg, sc4, w2r):
    return pl.pallas_call(
        _agg_body,
        grid=(B, N // AN),
        in_specs=[
            pl.BlockSpec((1, K, AN, CIN), lambda b, t: (b, 0, t, 0)),
            pl.BlockSpec((1, AN, K, M), lambda b, t: (b, t, 0, 0)),
            pl.BlockSpec((M, CIN, COUT), lambda b, t: (0, 0, 0)),
        ],
        out_specs=[
            pl.BlockSpec((1, AN, COUT), lambda b, t: (b, t, 0)),
            pl.BlockSpec((2, COUT), lambda b, t: (0, 0)),
        ],
        out_shape=[
            jax.ShapeDtypeStruct((B, N, COUT), jnp.float32),
            jax.ShapeDtypeStruct((2, COUT), jnp.float32),
        ],
        scratch_shapes=[pltpu.VMEM((2, COUT), jnp.float32)],
    )(hg, sc4, w2r)


def _fin_body(o_ref, st_ref, g_ref, b_ref, out_ref):
    cnt = float(B * N)
    mean = st_ref[0:1] / cnt
    var = st_ref[1:2] / cnt - mean * mean
    scale = g_ref[...] * lax.rsqrt(var + EPS)
    shift = b_ref[...] - mean * scale
    z = jnp.maximum(o_ref[0] * scale + shift, 0.0)
    out_ref[0] = z.T


def _finalize(out_pre, st, g, b):
    return pl.pallas_call(
        _fin_body,
        grid=(B, N // FN),
        in_specs=[
            pl.BlockSpec((1, FN, COUT), lambda bb, t: (bb, t, 0)),
            pl.BlockSpec((2, COUT), lambda bb, t: (0, 0)),
            pl.BlockSpec((1, COUT), lambda bb, t: (0, 0)),
            pl.BlockSpec((1, COUT), lambda bb, t: (0, 0)),
        ],
        out_specs=pl.BlockSpec((1, COUT, FN), lambda bb, t: (bb, 0, t)),
        out_shape=jax.ShapeDtypeStruct((B, COUT, N), jnp.float32),
    )(out_pre, st, g.reshape(1, COUT), b.reshape(1, COUT))


def kernel(x, idx, xyz_score, conv1_w, bn1_g, bn1_b, sW0, sg0, sb0,
           sW1, sg1, sb1, sW2, sg2, sb2, sW3, sb3, matrice2, bn2_g, bn2_b):
    hT = _compute_hT(x, conv1_w, bn1_g, bn1_b)
    table = hT.reshape(B * N, CIN)
    gidx = _prep_idx(idx.astype(jnp.int32)).reshape(R)
    hg = _gather_rows(table, gidx).reshape(B, K, N, CIN)

    xyz = xyz_score.reshape(B, 66, NK)
    y0, st0 = _layer0(xyz, sW0)
    y1, st1 = _mid_layer(y0, st0, sg0, sb0, sW1)
    y2, st2 = _mid_layer(y1, st1, sg1, sb1, sW2)
    score = _last_layer(y2, st2, sg2, sb2, sW3, sb3)
    sc4 = score.reshape(B, N, K, M)

    w2r = matrice2.reshape(CIN, M, COUT).transpose(1, 0, 2)
    out_pre, st3 = _aggregate(hg, sc4, w2r)
    return _finalize(out_pre, st3, bn2_g, bn2_b)


# speedup vs baseline: 1.0158x; 1.0158x over previous
import jax
import jax.numpy as jnp
from jax import lax
from jax.experimental import pallas as pl
from jax.experimental.pallas import tpu as pltpu
from jax.experimental.pallas import tpu_sc as plsc

B, N, K = 8, 1024, 16
CIN, COUT, M = 64, 64, 8
NK = N * K
R = B * NK
EPS = 1e-5

TK = 4096
NT = NK // TK
AN = 256
FN = 512

SC_CORES = 2
SC_SUBCORES = 16
SC_WORKERS = SC_CORES * SC_SUBCORES
SC_CH = 128
PER_W = R // SC_WORKERS
SC_NCH = PER_W // SC_CH


def _h_body(x_ref, w_ref, g_ref, b_ref, out_ref):
    w = w_ref[...]
    ys = []
    s = jnp.zeros((CIN, 1), jnp.float32)
    q = jnp.zeros((CIN, 1), jnp.float32)
    for b in range(B):
        y = jnp.dot(w, x_ref[b], preferred_element_type=jnp.float32)
        ys.append(y)
        s = s + jnp.sum(y, axis=1, keepdims=True)
        q = q + jnp.sum(y * y, axis=1, keepdims=True)
    cnt = float(B * N)
    mean = s / cnt
    var = q / cnt - mean * mean
    scale = g_ref[...] * lax.rsqrt(var + EPS)
    shift = b_ref[...] - mean * scale
    for b in range(B):
        z = jnp.maximum(ys[b] * scale + shift, 0.0)
        out_ref[b] = z.T.astype(jnp.bfloat16)


def _compute_hT(x, conv1_w, bn1_g, bn1_b):
    return pl.pallas_call(
        _h_body,
        out_shape=jax.ShapeDtypeStruct((B, N, CIN), jnp.bfloat16),
    )(x, conv1_w, bn1_g.reshape(CIN, 1), bn1_b.reshape(CIN, 1))


def _idx_body(i_ref, o_ref):
    b = pl.program_id(0)
    o_ref[0] = i_ref[0].T + b * N


def _prep_idx(idx):
    return pl.pallas_call(
        _idx_body,
        grid=(B,),
        in_specs=[pl.BlockSpec((1, N, K), lambda b: (b, 0, 0))],
        out_specs=pl.BlockSpec((1, K, N), lambda b: (b, 0, 0)),
        out_shape=jax.ShapeDtypeStruct((B, K, N), jnp.int32),
    )(idx)


def _sc_gather_body(table_hbm, gidx_hbm, out_hbm, idx_v, rows_v, sem):
    wid = lax.axis_index("s") * SC_CORES + lax.axis_index("c")
    base = wid * PER_W

    def chunk(c, carry):
        off = base + c * SC_CH
        pltpu.sync_copy(gidx_hbm.at[pl.ds(off, SC_CH)], idx_v)
        pltpu.async_copy(table_hbm.at[idx_v], rows_v, sem).wait()
        pltpu.sync_copy(rows_v, out_hbm.at[pl.ds(off, SC_CH)])
        return carry

    lax.fori_loop(0, SC_NCH, chunk, 0)


def _gather_rows(table, gidx):
    mesh = plsc.VectorSubcoreMesh(
        core_axis_name="c", subcore_axis_name="s",
        num_cores=SC_CORES, num_subcores=SC_SUBCORES)
    fn = pl.kernel(
        _sc_gather_body,
        out_type=jax.ShapeDtypeStruct((R, CIN), jnp.bfloat16),
        mesh=mesh,
        compiler_params=pltpu.CompilerParams(use_tc_tiling_on_sc=False),
        scratch_types=[
            pltpu.VMEM((SC_CH,), jnp.int32),
            pltpu.VMEM((SC_CH, CIN), jnp.bfloat16),
            pltpu.SemaphoreType.DMA,
        ],
    )
    return fn(table, gidx)


def _layer0_body(s_ref, w_ref, y_ref, st_ref, acc_ref):
    t = pl.program_id(0) * pl.num_programs(1) + pl.program_id(1)
    y = jnp.dot(w_ref[...], s_ref[0], preferred_element_type=jnp.float32)
    y_ref[0] = y

    @pl.when(t == 0)
    def _():
        acc_ref[...] = jnp.zeros_like(acc_ref)

    acc_ref[...] += jnp.concatenate(
        [jnp.sum(y, axis=1, keepdims=True),
         jnp.sum(y * y, axis=1, keepdims=True)], axis=1)

    @pl.when(t == pl.num_programs(0) * pl.num_programs(1) - 1)
    def _():
        st_ref[...] = acc_ref[...]


def _mid_body(y_ref, st_ref, g_ref, b_ref, w_ref, o_ref, sto_ref, acc_ref):
    t = pl.program_id(0) * pl.num_programs(1) + pl.num_programs(1) * 0 + pl.program_id(1)
    cnt = float(R)
    st = st_ref[...]
    mean = st[:, 0:1] / cnt
    var = st[:, 1:2] / cnt - mean * mean
    scale = g_ref[...] * lax.rsqrt(var + EPS)
    shift = b_ref[...] - mean * scale
    z = jnp.maximum(y_ref[0] * scale + shift, 0.0)
    y = jnp.dot(w_ref[...], z, preferred_element_type=jnp.float32)
    o_ref[0] = y

    @pl.when(t == 0)
    def _():
        acc_ref[...] = jnp.zeros_like(acc_ref)

    acc_ref[...] += jnp.concatenate(
        [jnp.sum(y, axis=1, keepdims=True),
         jnp.sum(y * y, axis=1, keepdims=True)], axis=1)

    @pl.when(t == pl.num_programs(0) * pl.num_programs(1) - 1)
    def _():
        sto_ref[...] = acc_ref[...]


def _s4_body(y_ref, st_ref, g_ref, b_ref, w_ref, bias_ref, score_ref):
    cnt = float(R)
    st = st_ref[...]
    mean = st[:, 0:1] / cnt
    var = st[:, 1:2] / cnt - mean * mean
    scale = g_ref[...] * lax.rsqrt(var + EPS)
    shift = b_ref[...] - mean * scale
    z = jnp.maximum(y_ref[0] * scale + shift, 0.0)
    y3 = jnp.dot(w_ref[...], z, preferred_element_type=jnp.float32) + bias_ref[...]
    mx = jnp.max(y3, axis=0, keepdims=True)
    e = jnp.exp(y3 - mx)
    sm = e / jnp.sum(e, axis=0, keepdims=True)
    score_ref[0] = sm.T


def _layer0(xyz, w):
    ci, co = w.shape[1], w.shape[0]
    return pl.pallas_call(
        _layer0_body,
        grid=(B, NT),
        in_specs=[
            pl.BlockSpec((1, ci, TK), lambda b, t: (b, 0, t)),
            pl.BlockSpec((co, ci), lambda b, t: (0, 0)),
        ],
        out_specs=[
            pl.BlockSpec((1, co, TK), lambda b, t: (b, 0, t)),
            pl.BlockSpec((co, 2), lambda b, t: (0, 0)),
        ],
        out_shape=[
            jax.ShapeDtypeStruct((B, co, NK), jnp.float32),
            jax.ShapeDtypeStruct((co, 2), jnp.float32),
        ],
        scratch_shapes=[pltpu.VMEM((co, 2), jnp.float32)],
    )(xyz, w)


def _mid_layer(y, st, g, b, w):
    ci, co = w.shape[1], w.shape[0]
    return pl.pallas_call(
        _mid_body,
        grid=(B, NT),
        in_specs=[
            pl.BlockSpec((1, ci, TK), lambda bb, t: (bb, 0, t)),
            pl.BlockSpec((ci, 2), lambda bb, t: (0, 0)),
            pl.BlockSpec((ci, 1), lambda bb, t: (0, 0)),
            pl.BlockSpec((ci, 1), lambda bb, t: (0, 0)),
            pl.BlockSpec((co, ci), lambda bb, t: (0, 0)),
        ],
        out_specs=[
            pl.BlockSpec((1, co, TK), lambda bb, t: (bb, 0, t)),
            pl.BlockSpec((co, 2), lambda bb, t: (0, 0)),
        ],
        out_shape=[
            jax.ShapeDtypeStruct((B, co, NK), jnp.float32),
            jax.ShapeDtypeStruct((co, 2), jnp.float32),
        ],
        scratch_shapes=[pltpu.VMEM((co, 2), jnp.float32)],
    )(y, st, g.reshape(ci, 1), b.reshape(ci, 1), w)


def _last_layer(y, st, g, b, w, bias):
    ci, co = w.shape[1], w.shape[0]
    return pl.pallas_call(
        _s4_body,
        grid=(B, NT),
        in_specs=[
            pl.BlockSpec((1, ci, TK), lambda bb, t: (bb, 0, t)),
            pl.BlockSpec((ci, 2), lambda bb, t: (0, 0)),
            pl.BlockSpec((ci, 1), lambda bb, t: (0, 0)),
            pl.BlockSpec((ci, 1), lambda bb, t: (0, 0)),
            pl.BlockSpec((co, ci), lambda bb, t: (0, 0)),
            pl.BlockSpec((co, 1), lambda bb, t: (0, 0)),
        ],
        out_specs=pl.BlockSpec((1, TK, co), lambda bb, t: (bb, t, 0)),
        out_shape=jax.ShapeDtypeStruct((B, NK, co), jnp.float32),
    )(y, st, g.reshape(ci, 1), b.reshape(ci, 1), w, bias.reshape(co, 1))


def _agg_body(hg_ref, sc_ref, w2_ref, o_ref, st_ref, acc_ref):
    t = pl.program_id(0) * pl.num_programs(1) + pl.program_id(1)
    hgv = hg_ref[0]
    scv = jnp.transpose(sc_ref[0], (1, 0, 2))
    acc = jnp.zeros((AN, COUT), jnp.float32)
    for m in range(M):
        w = scv[:, :, m:m + 1]
        am = jnp.sum(hgv * w, axis=0)

        acc = acc + jnp.dot(am, w2_ref[m], preferred_element_type=jnp.float32)
    o_ref[0] = acc

    @pl.when(t == 0)
    def _():
        acc_ref[...] = jnp.zeros_like(acc_ref)

    acc_ref[...] += jnp.concatenate(
        [jnp.sum(acc, axis=0, keepdims=True),
         jnp.sum(acc * acc, axis=0, keepdims=True)], axis=0)

    @pl.when(t == pl.num_programs(0) * pl.num_programs(1) - 1)
    def _():
        st_ref[...] = acc_ref[...]


def _aggregate(hg, sc4, w2r):
    return pl.pallas_call(
        _agg_body,
        grid=(B, N // AN),
        in_specs=[
            pl.BlockSpec((1, K, AN, CIN), lambda b, t: (b, 0, t, 0)),
            pl.BlockSpec((1, AN, K, M), lambda b, t: (b, t, 0, 0)),
            pl.BlockSpec((M, CIN, COUT), lambda b, t: (0, 0, 0)),
        ],
        out_specs=[
            pl.BlockSpec((1, AN, COUT), lambda b, t: (b, t, 0)),
            pl.BlockSpec((2, COUT), lambda b, t: (0, 0)),
        ],
        out_shape=[
            jax.ShapeDtypeStruct((B, N, COUT), jnp.float32),
            jax.ShapeDtypeStruct((2, COUT), jnp.float32),
        ],
        scratch_shapes=[pltpu.VMEM((2, COUT), jnp.float32)],
    )(hg, sc4, w2r)


def _fin_body(o_ref, st_ref, g_ref, b_ref, out_ref):
    cnt = float(B * N)
    mean = st_ref[0:1] / cnt
    var = st_ref[1:2] / cnt - mean * mean
    scale = g_ref[...] * lax.rsqrt(var + EPS)
    shift = b_ref[...] - mean * scale
    z = jnp.maximum(o_ref[0] * scale + shift, 0.0)
    out_ref[0] = z.T


def _finalize(out_pre, st, g, b):
    return pl.pallas_call(
        _fin_body,
        grid=(B, N // FN),
        in_specs=[
            pl.BlockSpec((1, FN, COUT), lambda bb, t: (bb, t, 0)),
            pl.BlockSpec((2, COUT), lambda bb, t: (0, 0)),
            pl.BlockSpec((1, COUT), lambda bb, t: (0, 0)),
            pl.BlockSpec((1, COUT), lambda bb, t: (0, 0)),
        ],
        out_specs=pl.BlockSpec((1, COUT, FN), lambda bb, t: (bb, 0, t)),
        out_shape=jax.ShapeDtypeStruct((B, COUT, N), jnp.float32),
    )(out_pre, st, g.reshape(1, COUT), b.reshape(1, COUT))


def kernel(x, idx, xyz_score, conv1_w, bn1_g, bn1_b, sW0, sg0, sb0,
           sW1, sg1, sb1, sW2, sg2, sb2, sW3, sb3, matrice2, bn2_g, bn2_b):
    hT = _compute_hT(x, conv1_w, bn1_g, bn1_b)
    table = hT.reshape(B * N, CIN)
    gidx = _prep_idx(idx.astype(jnp.int32)).reshape(R)
    hg = _gather_rows(table, gidx).reshape(B, K, N, CIN)

    xyz = xyz_score.reshape(B, 66, NK)
    y0, st0 = _layer0(xyz, sW0)
    y1, st1 = _mid_layer(y0, st0, sg0, sb0, sW1)
    y2, st2 = _mid_layer(y1, st1, sg1, sb1, sW2)
    score = _last_layer(y2, st2, sg2, sb2, sW3, sb3)
    sc4 = score.reshape(B, N, K, M)

    w2r = matrice2.reshape(CIN, M, COUT).transpose(1, 0, 2)
    out_pre, st3 = _aggregate(hg, sc4, w2r)
    return _finalize(out_pre, st3, bn2_g, bn2_b)


# speedup vs baseline: 1.0835x; 1.0667x over previous
import jax
import jax.numpy as jnp
from jax import lax
from jax.experimental import pallas as pl
from jax.experimental.pallas import tpu as pltpu
from jax.experimental.pallas import tpu_sc as plsc

B, N, K = 8, 1024, 16
CIN, COUT, M = 64, 64, 8
NK = N * K
R = B * NK
EPS = 1e-5

TK = 4096
NT = NK // TK
AN = 256
FN = 512

SC_CORES = 2
SC_SUBCORES = 16
SC_WORKERS = SC_CORES * SC_SUBCORES
SC_CH = 128
PER_W = R // SC_WORKERS
SC_NCH = PER_W // SC_CH


def _h_body(x_ref, w_ref, g_ref, b_ref, out_ref):
    w = w_ref[...]
    ys = []
    s = jnp.zeros((CIN, 1), jnp.float32)
    q = jnp.zeros((CIN, 1), jnp.float32)
    for b in range(B):
        y = jnp.dot(w, x_ref[b], preferred_element_type=jnp.float32)
        ys.append(y)
        s = s + jnp.sum(y, axis=1, keepdims=True)
        q = q + jnp.sum(y * y, axis=1, keepdims=True)
    cnt = float(B * N)
    mean = s / cnt
    var = q / cnt - mean * mean
    scale = g_ref[...] * lax.rsqrt(var + EPS)
    shift = b_ref[...] - mean * scale
    for b in range(B):
        z = jnp.maximum(ys[b] * scale + shift, 0.0)
        out_ref[b] = z.T


def _compute_hT(x, conv1_w, bn1_g, bn1_b):
    return pl.pallas_call(
        _h_body,
        out_shape=jax.ShapeDtypeStruct((B, N, CIN), jnp.float32),
    )(x, conv1_w, bn1_g.reshape(CIN, 1), bn1_b.reshape(CIN, 1))


def _idx_body(i_ref, o_ref):
    b = pl.program_id(0)
    o_ref[0] = i_ref[0].T + b * N


def _prep_idx(idx):
    return pl.pallas_call(
        _idx_body,
        grid=(B,),
        in_specs=[pl.BlockSpec((1, N, K), lambda b: (b, 0, 0))],
        out_specs=pl.BlockSpec((1, K, N), lambda b: (b, 0, 0)),
        out_shape=jax.ShapeDtypeStruct((B, K, N), jnp.int32),
    )(idx)


def _sc_gather_body(table_hbm, gidx_hbm, out_hbm, idx_v, rows_v, sem):
    wid = lax.axis_index("s") * SC_CORES + lax.axis_index("c")
    base = wid * PER_W

    def chunk(c, carry):
        off = base + c * SC_CH
        pltpu.sync_copy(gidx_hbm.at[pl.ds(off, SC_CH)], idx_v)
        pltpu.async_copy(table_hbm.at[idx_v], rows_v, sem).wait()
        pltpu.sync_copy(rows_v, out_hbm.at[pl.ds(off, SC_CH)])
        return carry

    lax.fori_loop(0, SC_NCH, chunk, 0)


def _gather_rows(table, gidx):
    mesh = plsc.VectorSubcoreMesh(
        core_axis_name="c", subcore_axis_name="s",
        num_cores=SC_CORES, num_subcores=SC_SUBCORES)
    fn = pl.kernel(
        _sc_gather_body,
        out_type=jax.ShapeDtypeStruct((R, CIN), jnp.float32),
        mesh=mesh,
        compiler_params=pltpu.CompilerParams(use_tc_tiling_on_sc=False),
        scratch_types=[
            pltpu.VMEM((SC_CH,), jnp.int32),
            pltpu.VMEM((SC_CH, CIN), jnp.float32),
            pltpu.SemaphoreType.DMA,
        ],
    )
    return fn(table, gidx)


def _layer0_body(s_ref, w_ref, y_ref, st_ref, acc_ref):
    t = pl.program_id(0) * pl.num_programs(1) + pl.program_id(1)
    y = jnp.dot(w_ref[...], s_ref[0], preferred_element_type=jnp.float32)
    y_ref[0] = y

    @pl.when(t == 0)
    def _():
        acc_ref[...] = jnp.zeros_like(acc_ref)

    acc_ref[...] += jnp.concatenate(
        [jnp.sum(y, axis=1, keepdims=True),
         jnp.sum(y * y, axis=1, keepdims=True)], axis=1)

    @pl.when(t == pl.num_programs(0) * pl.num_programs(1) - 1)
    def _():
        st_ref[...] = acc_ref[...]


def _mid_body(y_ref, st_ref, g_ref, b_ref, w_ref, o_ref, sto_ref, acc_ref):
    t = pl.program_id(0) * pl.num_programs(1) + pl.num_programs(1) * 0 + pl.program_id(1)
    cnt = float(R)
    st = st_ref[...]
    mean = st[:, 0:1] / cnt
    var = st[:, 1:2] / cnt - mean * mean
    scale = g_ref[...] * lax.rsqrt(var + EPS)
    shift = b_ref[...] - mean * scale
    z = jnp.maximum(y_ref[0] * scale + shift, 0.0)
    y = jnp.dot(w_ref[...], z, preferred_element_type=jnp.float32)
    o_ref[0] = y

    @pl.when(t == 0)
    def _():
        acc_ref[...] = jnp.zeros_like(acc_ref)

    acc_ref[...] += jnp.concatenate(
        [jnp.sum(y, axis=1, keepdims=True),
         jnp.sum(y * y, axis=1, keepdims=True)], axis=1)

    @pl.when(t == pl.num_programs(0) * pl.num_programs(1) - 1)
    def _():
        sto_ref[...] = acc_ref[...]


def _s4_body(y_ref, st_ref, g_ref, b_ref, w_ref, bias_ref, score_ref):
    cnt = float(R)
    st = st_ref[...]
    mean = st[:, 0:1] / cnt
    var = st[:, 1:2] / cnt - mean * mean
    scale = g_ref[...] * lax.rsqrt(var + EPS)
    shift = b_ref[...] - mean * scale
    z = jnp.maximum(y_ref[0] * scale + shift, 0.0)
    y3 = jnp.dot(w_ref[...], z, preferred_element_type=jnp.float32) + bias_ref[...]
    mx = jnp.max(y3, axis=0, keepdims=True)
    e = jnp.exp(y3 - mx)
    sm = e / jnp.sum(e, axis=0, keepdims=True)
    score_ref[0] = sm.T


def _layer0(xyz, w):
    ci, co = w.shape[1], w.shape[0]
    return pl.pallas_call(
        _layer0_body,
        grid=(B, NT),
        in_specs=[
            pl.BlockSpec((1, ci, TK), lambda b, t: (b, 0, t)),
            pl.BlockSpec((co, ci), lambda b, t: (0, 0)),
        ],
        out_specs=[
            pl.BlockSpec((1, co, TK), lambda b, t: (b, 0, t)),
            pl.BlockSpec((co, 2), lambda b, t: (0, 0)),
        ],
        out_shape=[
            jax.ShapeDtypeStruct((B, co, NK), jnp.float32),
            jax.ShapeDtypeStruct((co, 2), jnp.float32),
        ],
        scratch_shapes=[pltpu.VMEM((co, 2), jnp.float32)],
    )(xyz, w)


def _mid_layer(y, st, g, b, w):
    ci, co = w.shape[1], w.shape[0]
    return pl.pallas_call(
        _mid_body,
        grid=(B, NT),
        in_specs=[
            pl.BlockSpec((1, ci, TK), lambda bb, t: (bb, 0, t)),
            pl.BlockSpec((ci, 2), lambda bb, t: (0, 0)),
            pl.BlockSpec((ci, 1), lambda bb, t: (0, 0)),
            pl.BlockSpec((ci, 1), lambda bb, t: (0, 0)),
            pl.BlockSpec((co, ci), lambda bb, t: (0, 0)),
        ],
        out_specs=[
            pl.BlockSpec((1, co, TK), lambda bb, t: (bb, 0, t)),
            pl.BlockSpec((co, 2), lambda bb, t: (0, 0)),
        ],
        out_shape=[
            jax.ShapeDtypeStruct((B, co, NK), jnp.float32),
            jax.ShapeDtypeStruct((co, 2), jnp.float32),
        ],
        scratch_shapes=[pltpu.VMEM((co, 2), jnp.float32)],
    )(y, st, g.reshape(ci, 1), b.reshape(ci, 1), w)


def _last_layer(y, st, g, b, w, bias):
    ci, co = w.shape[1], w.shape[0]
    return pl.pallas_call(
        _s4_body,
        grid=(B, NT),
        in_specs=[
            pl.BlockSpec((1, ci, TK), lambda bb, t: (bb, 0, t)),
            pl.BlockSpec((ci, 2), lambda bb, t: (0, 0)),
            pl.BlockSpec((ci, 1), lambda bb, t: (0, 0)),
            pl.BlockSpec((ci, 1), lambda bb, t: (0, 0)),
            pl.BlockSpec((co, ci), lambda bb, t: (0, 0)),
            pl.BlockSpec((co, 1), lambda bb, t: (0, 0)),
        ],
        out_specs=pl.BlockSpec((1, TK, co), lambda bb, t: (bb, t, 0)),
        out_shape=jax.ShapeDtypeStruct((B, NK, co), jnp.float32),
    )(y, st, g.reshape(ci, 1), b.reshape(ci, 1), w, bias.reshape(co, 1))


def _agg_body(hg_ref, sc_ref, w2_ref, o_ref, st_ref, acc_ref):
    t = pl.program_id(0) * pl.num_programs(1) + pl.program_id(1)
    hgv = hg_ref[0]
    scv = jnp.transpose(sc_ref[0], (1, 0, 2))
    acc = jnp.zeros((AN, COUT), jnp.float32)
    for m in range(M):
        w = scv[:, :, m:m + 1]
        am = jnp.sum(hgv * w, axis=0)

        acc = acc + jnp.dot(am, w2_ref[m], preferred_element_type=jnp.float32)
    o_ref[0] = acc

    @pl.when(t == 0)
    def _():
        acc_ref[...] = jnp.zeros_like(acc_ref)

    acc_ref[...] += jnp.concatenate(
        [jnp.sum(acc, axis=0, keepdims=True),
         jnp.sum(acc * acc, axis=0, keepdims=True)], axis=0)

    @pl.when(t == pl.num_programs(0) * pl.num_programs(1) - 1)
    def _():
        st_ref[...] = acc_ref[...]


def _aggregate(hg, sc4, w2r):
    return pl.pallas_call(
        _agg_body,
        grid=(B, N // AN),
        in_specs=[
            pl.BlockSpec((1, K, AN, CIN), lambda b, t: (b, 0, t, 0)),
            pl.BlockSpec((1, AN, K, M), lambda b, t: (b, t, 0, 0)),
            pl.BlockSpec((M, CIN, COUT), lambda b, t: (0, 0, 0)),
        ],
        out_specs=[
            pl.BlockSpec((1, AN, COUT), lambda b, t: (b, t, 0)),
            pl.BlockSpec((2, COUT), lambda b, t: (0, 0)),
        ],
        out_shape=[
            jax.ShapeDtypeStruct((B, N, COUT), jnp.float32),
            jax.ShapeDtypeStruct((2, COUT), jnp.float32),
        ],
        scratch_shapes=[pltpu.VMEM((2, COUT), jnp.float32)],
    )(hg, sc4, w2r)


def _fin_body(o_ref, st_ref, g_ref, b_ref, out_ref):
    cnt = float(B * N)
    mean = st_ref[0:1] / cnt
    var = st_ref[1:2] / cnt - mean * mean
    scale = g_ref[...] * lax.rsqrt(var + EPS)
    shift = b_ref[...] - mean * scale
    z = jnp.maximum(o_ref[0] * scale + shift, 0.0)
    out_ref[0] = z.T


def _finalize(out_pre, st, g, b):
    return pl.pallas_call(
        _fin_body,
        grid=(B, N // FN),
        in_specs=[
            pl.BlockSpec((1, FN, COUT), lambda bb, t: (bb, t, 0)),
            pl.BlockSpec((2, COUT), lambda bb, t: (0, 0)),
            pl.BlockSpec((1, COUT), lambda bb, t: (0, 0)),
            pl.BlockSpec((1, COUT), lambda bb, t: (0, 0)),
        ],
        out_specs=pl.BlockSpec((1, COUT, FN), lambda bb, t: (bb, 0, t)),
        out_shape=jax.ShapeDtypeStruct((B, COUT, N), jnp.float32),
    )(out_pre, st, g.reshape(1, COUT), b.reshape(1, COUT))


def kernel(x, idx, xyz_score, conv1_w, bn1_g, bn1_b, sW0, sg0, sb0,
           sW1, sg1, sb1, sW2, sg2, sb2, sW3, sb3, matrice2, bn2_g, bn2_b):
    hT = _compute_hT(x, conv1_w, bn1_g, bn1_b)
    table = hT.reshape(B * N, CIN)
    gidx = _prep_idx(idx.astype(jnp.int32)).reshape(R)
    hg = _gather_rows(table, gidx).reshape(B, K, N, CIN)

    xyz = xyz_score.reshape(B, 66, NK)
    y0, st0 = _layer0(xyz, sW0)
    y1, st1 = _mid_layer(y0, st0, sg0, sb0, sW1)
    y2, st2 = _mid_layer(y1, st1, sg1, sb1, sW2)
    score = _last_layer(y2, st2, sg2, sb2, sW3, sb3)
    sc4 = score.reshape(B, N, K, M)

    w2r = matrice2.reshape(CIN, M, COUT).transpose(1, 0, 2)
    out_pre, st3 = _aggregate(hg, sc4, w2r)
    return _finalize(out_pre, st3, bn2_g, bn2_b)
